# baseline (speedup 1.0000x reference)
import jax
import jax.numpy as jnp
from jax import lax
from jax.experimental import pallas as pl
from jax.experimental.pallas import tpu as pltpu
from jax.experimental.pallas import tpu_sc as plsc

DIM = 16
BATCH = 16384
NC = 2
NS = 16
L = 16
NW = NC * NS
BPW = BATCH // NW
NGRP = BPW // L
ROWS_PER_GRP = 2 * DIM * L // 128
DBLOCK = 8000512


def _loss_body(x_hbm, deg_hbm, tab_hbm, out_hbm, idx_v, deg_v, abuf_v,
               dbuf_v, res_v, sem):
    wid = lax.axis_index("s") * NC + lax.axis_index("c")
    pltpu.sync_copy(x_hbm.at[wid], idx_v)
    pltpu.sync_copy(deg_hbm.at[wid], deg_v)

    iota = lax.iota(jnp.int32, L)

    for g in range(NGRP):
        k, lp = divmod(g, 4)
        kv = jnp.full((L,), k, jnp.int32)
        p0 = lp * 2 * L + iota * 2
        v0 = plsc.load_gather(idx_v, [kv, p0])
        v1 = plsc.load_gather(idx_v, [kv, p0 + 1])
        for side, b in ((0, v0), (1, v1)):
            for d in range(DIM):
                j = (g * 2 + side) * DIM + d
                abuf_v[j >> 3, pl.ds((j & 7) * L, L)] = b + d * 1000000

    copies = [
        pltpu.async_copy(tab_hbm.at[abuf_v.at[j]], dbuf_v.at[j], sem)
        for j in range(128)
    ]

    def sqrt16(s):
        i = plsc.bitcast(s, jnp.int32)
        i = jnp.int32(0x5F3759DF) - (i >> 1)
        y = plsc.bitcast(i, jnp.float32)
        for _ in range(3):
            y = y * (1.5 - 0.5 * s * y * y)
        return jnp.where(s > 0.0, s * y, 0.0)

    acc = jnp.zeros((L,), jnp.float32)
    for g in range(NGRP):
        for r in range(ROWS_PER_GRP):
            copies[g * ROWS_PER_GRP + r].wait()
        s = jnp.zeros((L,), jnp.float32)
        for d in range(DIM):
            a = dbuf_v[g * 4 + (d >> 3), pl.ds((d & 7) * L, L)]
            b = dbuf_v[g * 4 + 2 + (d >> 3), pl.ds((d & 7) * L, L)]
            df = a - b
            s = s + df * df
        dist = sqrt16(s)
        deg = deg_v[pl.ds(g * L, L)]
        acc = acc + deg * dist + jnp.exp(-dist)

    res_v[...] = jnp.full((L,), jnp.sum(acc), jnp.float32)
    pltpu.sync_copy(res_v, out_hbm.at[wid])


def kernel(x, degrees, embs):
    tab = embs.T.reshape(DIM * 1000000)
    xr = x.astype(jnp.int32).reshape(NW, 8, 128)
    dr = degrees.reshape(NW, BPW)
    mesh = plsc.VectorSubcoreMesh(core_axis_name="c", subcore_axis_name="s")
    out = pl.kernel(
        _loss_body,
        mesh=mesh,
        out_type=jax.ShapeDtypeStruct((NW, L), jnp.float32),
        scratch_types=[
            pltpu.VMEM((8, 128), jnp.int32),
            pltpu.VMEM((BPW,), jnp.float32),
            pltpu.VMEM((128, 128), jnp.int32),
            pltpu.VMEM((128, 128), jnp.float32),
            pltpu.VMEM((L,), jnp.float32),
            pltpu.SemaphoreType.DMA,
        ],
        compiler_params=pltpu.CompilerParams(needs_layout_passes=False,
                                             use_tc_tiling_on_sc=False),
    )(xr, dr, tab)
    return jnp.sum(out[:, 0])


# speedup vs baseline: 7.3022x; 7.3022x over previous
import jax
import jax.numpy as jnp
from jax import lax
from jax.experimental import pallas as pl
from jax.experimental.pallas import tpu as pltpu
from jax.experimental.pallas import tpu_sc as plsc

DIM = 16
BATCH = 16384
NC = 2
NS = 16
L = 16
NW = NC * NS
BPW = BATCH // NW
NGRP = BPW // L
ROWS_PER_GRP = 2 * DIM * L // 128
DBLOCK = 8000512


NFULL = 1000000 // 128
KCOL = 14
NBLK = 18
TILE_W = 2048
WBLK = KCOL * TILE_W


def _detile_body(embs_t_hbm, tail_t_hbm, out_hbm, rbuf0_v, rbuf1_v,
                 wbuf0_v, wbuf1_v, tail_v, sem_r0, sem_r1, sem_w0, sem_w1):
    wid = lax.axis_index("s") * NC + lax.axis_index("c")
    rbufs = (rbuf0_v, rbuf1_v)
    wbufs = (wbuf0_v, wbuf1_v)
    rsems = (sem_r0, sem_r1)
    wsems = (sem_w0, sem_w1)

    def tile0(blk):
        return jnp.minimum((wid * NBLK + blk) * KCOL, NFULL - KCOL)

    def fire_reads_dyn(blk, par):
        t0 = tile0(blk)
        for j in range(KCOL):
            pltpu.async_copy(
                embs_t_hbm.at[:, pl.ds((t0 + j) * 128, 128)],
                rbufs[par].at[:, pl.ds(j * 128, 128)], rsems[par])

    def fire_reads(blk):
        fire_reads_dyn(blk, blk % 2)

    def drain(ref, sem):
        pltpu.make_async_copy(out_hbm.at[pl.ds(0, ref.shape[0])]
                              if len(ref.shape) == 1 else
                              embs_t_hbm.at[:, pl.ds(0, ref.shape[1])],
                              ref, sem).wait()

    fire_reads(0)

    def pair_body(pair, carry):
        for b in (0, 1):
            blk = pair * 2 + b
            drain(rbufs[b], rsems[b])

            @pl.when(blk + 1 < NBLK)
            def _():
                fire_reads_dyn(blk + 1, 1 - b)

            @pl.when(blk >= 2)
            def _():
                drain(wbufs[b], wsems[b])

            rbuf = rbufs[b]
            wbuf = wbufs[b]

            def repack(j, c2):
                for d in range(DIM):
                    for c in range(128 // L):
                        wbuf[pl.ds(j * TILE_W + d * 128 + c * L, L)] = (
                            rbuf[d, pl.ds(j * 128 + c * L, L)])
                return c2

            lax.fori_loop(0, KCOL, repack, jnp.int32(0))
            pltpu.async_copy(wbuf,
                             out_hbm.at[pl.ds(tile0(blk) * TILE_W, WBLK)],
                             wsems[b])
        return carry

    lax.fori_loop(0, NBLK // 2, pair_body, jnp.int32(0))
    drain(wbufs[0], wsems[0])
    drain(wbufs[1], wsems[1])

    @pl.when(wid == 0)
    def _tail():
        pltpu.sync_copy(tail_t_hbm, tail_v)
        for d in range(DIM):
            for c in range(64 // L):
                wbuf0_v[pl.ds(d * 128 + c * L, L)] = tail_v[d, pl.ds(c * L, L)]
        pltpu.sync_copy(wbuf0_v.at[pl.ds(0, TILE_W)],
                        out_hbm.at[pl.ds(NFULL * TILE_W, TILE_W)])


def _loss_body(x_hbm, deg_hbm, tab_hbm, out_hbm, idx_v, deg_v, abuf_v,
               dbuf_v, res_v, sem):
    wid = lax.axis_index("s") * NC + lax.axis_index("c")
    pltpu.sync_copy(x_hbm.at[wid], idx_v)
    pltpu.sync_copy(deg_hbm.at[wid], deg_v)

    iota = lax.iota(jnp.int32, L)

    for g in range(NGRP):
        k, lp = divmod(g, 4)
        kv = jnp.full((L,), k, jnp.int32)
        p0 = lp * 2 * L + iota * 2
        v0 = plsc.load_gather(idx_v, [kv, p0])
        v1 = plsc.load_gather(idx_v, [kv, p0 + 1])
        b0 = ((v0 >> 7) << 11) + (v0 & 127)
        b1 = ((v1 >> 7) << 11) + (v1 & 127)
        for side, b in ((0, b0), (1, b1)):
            for d in range(DIM):
                j = (g * 2 + side) * DIM + d
                abuf_v[j >> 3, pl.ds((j & 7) * L, L)] = b + d * 128

    copies = [
        pltpu.async_copy(tab_hbm.at[abuf_v.at[j]], dbuf_v.at[j], sem)
        for j in range(128)
    ]

    def sqrt16(s):
        i = plsc.bitcast(s, jnp.int32)
        i = jnp.int32(0x5F3759DF) - (i >> 1)
        y = plsc.bitcast(i, jnp.float32)
        for _ in range(3):
            y = y * (1.5 - 0.5 * s * y * y)
        return jnp.where(s > 0.0, s * y, 0.0)

    acc = jnp.zeros((L,), jnp.float32)
    for g in range(NGRP):
        for r in range(ROWS_PER_GRP):
            copies[g * ROWS_PER_GRP + r].wait()
        s = jnp.zeros((L,), jnp.float32)
        for d in range(DIM):
            a = dbuf_v[g * 4 + (d >> 3), pl.ds((d & 7) * L, L)]
            b = dbuf_v[g * 4 + 2 + (d >> 3), pl.ds((d & 7) * L, L)]
            df = a - b
            s = s + df * df
        dist = sqrt16(s)
        deg = deg_v[pl.ds(g * L, L)]
        acc = acc + deg * dist + jnp.exp(-dist)

    res_v[...] = jnp.full((L,), jnp.sum(acc), jnp.float32)
    pltpu.sync_copy(res_v, out_hbm.at[wid])


def kernel(x, degrees, embs):
    xr = x.astype(jnp.int32).reshape(NW, 8, 128)
    dr = degrees.reshape(NW, BPW)
    mesh = plsc.VectorSubcoreMesh(core_axis_name="c", subcore_axis_name="s")
    tab = pl.kernel(
        _detile_body,
        mesh=mesh,
        out_type=jax.ShapeDtypeStruct(((NFULL + 1) * TILE_W,), jnp.float32),
        scratch_types=[
            pltpu.VMEM((DIM, KCOL * 128), jnp.float32),
            pltpu.VMEM((DIM, KCOL * 128), jnp.float32),
            pltpu.VMEM((WBLK,), jnp.float32),
            pltpu.VMEM((WBLK,), jnp.float32),
            pltpu.VMEM((DIM, 64), jnp.float32),
            pltpu.SemaphoreType.DMA,
            pltpu.SemaphoreType.DMA,
            pltpu.SemaphoreType.DMA,
            pltpu.SemaphoreType.DMA,
        ],
        compiler_params=pltpu.CompilerParams(needs_layout_passes=False),
    )(embs.T, embs.T[:, NFULL * 128:])
    out = pl.kernel(
        _loss_body,
        mesh=mesh,
        out_type=jax.ShapeDtypeStruct((NW, L), jnp.float32),
        scratch_types=[
            pltpu.VMEM((8, 128), jnp.int32),
            pltpu.VMEM((BPW,), jnp.float32),
            pltpu.VMEM((128, 128), jnp.int32),
            pltpu.VMEM((128, 128), jnp.float32),
            pltpu.VMEM((L,), jnp.float32),
            pltpu.SemaphoreType.DMA,
        ],
        compiler_params=pltpu.CompilerParams(needs_layout_passes=False,
                                             use_tc_tiling_on_sc=False),
    )(xr, dr, tab)
    return jnp.sum(out[:, 0])


# speedup vs baseline: 7.4866x; 1.0253x over previous
import jax
import jax.numpy as jnp
from jax import lax
from jax.experimental import pallas as pl
from jax.experimental.pallas import tpu as pltpu
from jax.experimental.pallas import tpu_sc as plsc

DIM = 16
BATCH = 16384
NC = 2
NS = 16
L = 16
NW = NC * NS
BPW = BATCH // NW
NGRP = BPW // L
ROWS_PER_GRP = 2 * DIM * L // 128
DBLOCK = 8000512


NFULL = 1000000 // 128
KCOL = 14
NBLK = 18
TILE_W = 2048
WBLK = KCOL * TILE_W


def _detile_body(embs_t_hbm, tail_t_hbm, out_hbm, rbuf0_v, rbuf1_v,
                 wbuf0_v, wbuf1_v, tail_v, sem_r0, sem_r1, sem_w0, sem_w1):
    wid = lax.axis_index("s") * NC + lax.axis_index("c")
    rbufs = (rbuf0_v, rbuf1_v)
    wbufs = (wbuf0_v, wbuf1_v)
    rsems = (sem_r0, sem_r1)
    wsems = (sem_w0, sem_w1)

    def tile0(blk):
        return jnp.minimum((wid * NBLK + blk) * KCOL, NFULL - KCOL)

    def fire_reads_dyn(blk, par):
        pltpu.async_copy(
            embs_t_hbm.at[:, pl.ds(tile0(blk) * 128, KCOL * 128)],
            rbufs[par], rsems[par])

    def fire_reads(blk):
        fire_reads_dyn(blk, blk % 2)

    def drain(ref, sem):
        pltpu.make_async_copy(out_hbm.at[pl.ds(0, ref.shape[0])]
                              if len(ref.shape) == 1 else
                              embs_t_hbm.at[:, pl.ds(0, ref.shape[1])],
                              ref, sem).wait()

    fire_reads(0)

    def pair_body(pair, carry):
        for b in (0, 1):
            blk = pair * 2 + b
            drain(rbufs[b], rsems[b])

            @pl.when(blk + 1 < NBLK)
            def _():
                fire_reads_dyn(blk + 1, 1 - b)

            @pl.when(blk >= 2)
            def _():
                drain(wbufs[b], wsems[b])

            rbuf = rbufs[b]
            wbuf = wbufs[b]

            def repack(j, c2):
                for d in range(DIM):
                    for c in range(128 // L):
                        wbuf[pl.ds(j * TILE_W + d * 128 + c * L, L)] = (
                            rbuf[d, pl.ds(j * 128 + c * L, L)])
                return c2

            lax.fori_loop(0, KCOL, repack, jnp.int32(0), unroll=2)
            pltpu.async_copy(wbuf,
                             out_hbm.at[pl.ds(tile0(blk) * TILE_W, WBLK)],
                             wsems[b])
        return carry

    lax.fori_loop(0, NBLK // 2, pair_body, jnp.int32(0))
    drain(wbufs[0], wsems[0])
    drain(wbufs[1], wsems[1])

    @pl.when(wid == 0)
    def _tail():
        pltpu.sync_copy(tail_t_hbm, tail_v)
        for d in range(DIM):
            for c in range(64 // L):
                wbuf0_v[pl.ds(d * 128 + c * L, L)] = tail_v[d, pl.ds(c * L, L)]
        pltpu.sync_copy(wbuf0_v.at[pl.ds(0, TILE_W)],
                        out_hbm.at[pl.ds(NFULL * TILE_W, TILE_W)])


def _loss_body(x_hbm, deg_hbm, tab_hbm, out_hbm, idx_v, deg_v, abuf_v,
               dbuf_v, res_v, sem):
    wid = lax.axis_index("s") * NC + lax.axis_index("c")
    pltpu.sync_copy(x_hbm.at[wid], idx_v)
    pltpu.sync_copy(deg_hbm.at[wid], deg_v)

    iota = lax.iota(jnp.int32, L)

    for g in range(NGRP):
        k, lp = divmod(g, 4)
        kv = jnp.full((L,), k, jnp.int32)
        p0 = lp * 2 * L + iota * 2
        v0 = plsc.load_gather(idx_v, [kv, p0])
        v1 = plsc.load_gather(idx_v, [kv, p0 + 1])
        b0 = ((v0 >> 7) << 11) + (v0 & 127)
        b1 = ((v1 >> 7) << 11) + (v1 & 127)
        for side, b in ((0, b0), (1, b1)):
            for d in range(DIM):
                j = (g * 2 + side) * DIM + d
                abuf_v[j >> 3, pl.ds((j & 7) * L, L)] = b + d * 128

    copies = [
        pltpu.async_copy(tab_hbm.at[abuf_v.at[j]], dbuf_v.at[j], sem)
        for j in range(128)
    ]

    def sqrt16(s):
        i = plsc.bitcast(s, jnp.int32)
        i = jnp.int32(0x5F3759DF) - (i >> 1)
        y = plsc.bitcast(i, jnp.float32)
        for _ in range(3):
            y = y * (1.5 - 0.5 * s * y * y)
        return jnp.where(s > 0.0, s * y, 0.0)

    acc = jnp.zeros((L,), jnp.float32)
    for g in range(NGRP):
        for r in range(ROWS_PER_GRP):
            copies[g * ROWS_PER_GRP + r].wait()
        s = jnp.zeros((L,), jnp.float32)
        for d in range(DIM):
            a = dbuf_v[g * 4 + (d >> 3), pl.ds((d & 7) * L, L)]
            b = dbuf_v[g * 4 + 2 + (d >> 3), pl.ds((d & 7) * L, L)]
            df = a - b
            s = s + df * df
        dist = sqrt16(s)
        deg = deg_v[pl.ds(g * L, L)]
        acc = acc + deg * dist + jnp.exp(-dist)

    res_v[...] = jnp.full((L,), jnp.sum(acc), jnp.float32)
    pltpu.sync_copy(res_v, out_hbm.at[wid])


def kernel(x, degrees, embs):
    xr = x.astype(jnp.int32).reshape(NW, 8, 128)
    dr = degrees.reshape(NW, BPW)
    mesh = plsc.VectorSubcoreMesh(core_axis_name="c", subcore_axis_name="s")
    tab = pl.kernel(
        _detile_body,
        mesh=mesh,
        out_type=jax.ShapeDtypeStruct(((NFULL + 1) * TILE_W,), jnp.float32),
        scratch_types=[
            pltpu.VMEM((DIM, KCOL * 128), jnp.float32),
            pltpu.VMEM((DIM, KCOL * 128), jnp.float32),
            pltpu.VMEM((WBLK,), jnp.float32),
            pltpu.VMEM((WBLK,), jnp.float32),
            pltpu.VMEM((DIM, 64), jnp.float32),
            pltpu.SemaphoreType.DMA,
            pltpu.SemaphoreType.DMA,
            pltpu.SemaphoreType.DMA,
            pltpu.SemaphoreType.DMA,
        ],
        compiler_params=pltpu.CompilerParams(needs_layout_passes=False),
    )(embs.T, embs.T[:, NFULL * 128:])
    out = pl.kernel(
        _loss_body,
        mesh=mesh,
        out_type=jax.ShapeDtypeStruct((NW, L), jnp.float32),
        scratch_types=[
            pltpu.VMEM((8, 128), jnp.int32),
            pltpu.VMEM((BPW,), jnp.float32),
            pltpu.VMEM((128, 128), jnp.int32),
            pltpu.VMEM((128, 128), jnp.float32),
            pltpu.VMEM((L,), jnp.float32),
            pltpu.SemaphoreType.DMA,
        ],
        compiler_params=pltpu.CompilerParams(needs_layout_passes=False,
                                             use_tc_tiling_on_sc=False),
    )(xr, dr, tab)
    return jnp.sum(out[:, 0])


# speedup vs baseline: 7.6525x; 1.0221x over previous
import jax
import jax.numpy as jnp
from jax import lax
from jax.experimental import pallas as pl
from jax.experimental.pallas import tpu as pltpu
from jax.experimental.pallas import tpu_sc as plsc

DIM = 16
BATCH = 16384
NC = 2
NS = 16
L = 16
NW = NC * NS
BPW = BATCH // NW
NGRP = BPW // L
ROWS_PER_GRP = 2 * DIM * L // 128
DBLOCK = 8000512


NFULL = 1000000 // 128
KCOL = 14
NBLK = 18
TILE_W = 2048
WBLK = KCOL * TILE_W


def _detile_body(embs_t_hbm, tail_t_hbm, out_hbm, rbuf0_v, rbuf1_v,
                 wbuf0_v, wbuf1_v, tail_v, sem_r0, sem_r1, sem_w0, sem_w1):
    wid = lax.axis_index("s") * NC + lax.axis_index("c")
    rbufs = (rbuf0_v, rbuf1_v)
    wbufs = (wbuf0_v, wbuf1_v)
    rsems = (sem_r0, sem_r1)
    wsems = (sem_w0, sem_w1)

    def tile0(blk):
        return jnp.minimum((wid * NBLK + blk) * KCOL, NFULL - KCOL)

    def fire_reads_dyn(blk, par):
        pltpu.async_copy(
            embs_t_hbm.at[:, pl.ds(tile0(blk) * 128, KCOL * 128)],
            rbufs[par], rsems[par])

    def fire_reads(blk):
        fire_reads_dyn(blk, blk % 2)

    def drain(ref, sem):
        pltpu.make_async_copy(out_hbm.at[pl.ds(0, ref.shape[0])]
                              if len(ref.shape) == 1 else
                              embs_t_hbm.at[:, pl.ds(0, ref.shape[1])],
                              ref, sem).wait()

    fire_reads(0)

    def pair_body(pair, carry):
        for b in (0, 1):
            blk = pair * 2 + b
            drain(rbufs[b], rsems[b])

            @pl.when(blk + 1 < NBLK)
            def _():
                fire_reads_dyn(blk + 1, 1 - b)

            @pl.when(blk >= 2)
            def _():
                drain(wbufs[b], wsems[b])

            rbuf = rbufs[b]
            wbuf = wbufs[b]

            def repack(j, c2):
                for d in range(DIM):
                    for c in range(128 // L):
                        wbuf[pl.ds(j * TILE_W + d * 128 + c * L, L)] = (
                            rbuf[d, pl.ds(j * 128 + c * L, L)])
                return c2

            lax.fori_loop(0, KCOL, repack, jnp.int32(0), unroll=4)
            pltpu.async_copy(wbuf,
                             out_hbm.at[pl.ds(tile0(blk) * TILE_W, WBLK)],
                             wsems[b])
        return carry

    lax.fori_loop(0, NBLK // 2, pair_body, jnp.int32(0))
    drain(wbufs[0], wsems[0])
    drain(wbufs[1], wsems[1])

    @pl.when(wid == 0)
    def _tail():
        pltpu.sync_copy(tail_t_hbm, tail_v)
        for d in range(DIM):
            for c in range(64 // L):
                wbuf0_v[pl.ds(d * 128 + c * L, L)] = tail_v[d, pl.ds(c * L, L)]
        pltpu.sync_copy(wbuf0_v.at[pl.ds(0, TILE_W)],
                        out_hbm.at[pl.ds(NFULL * TILE_W, TILE_W)])


def _loss_body(x_hbm, deg_hbm, tab_hbm, out_hbm, idx_v, deg_v, abuf_v,
               dbuf_v, res_v, sem):
    wid = lax.axis_index("s") * NC + lax.axis_index("c")
    pltpu.sync_copy(x_hbm.at[wid], idx_v)
    pltpu.sync_copy(deg_hbm.at[wid], deg_v)

    iota = lax.iota(jnp.int32, L)

    for g in range(NGRP):
        k, lp = divmod(g, 4)
        kv = jnp.full((L,), k, jnp.int32)
        p0 = lp * 2 * L + iota * 2
        v0 = plsc.load_gather(idx_v, [kv, p0])
        v1 = plsc.load_gather(idx_v, [kv, p0 + 1])
        b0 = ((v0 >> 7) << 11) + (v0 & 127)
        b1 = ((v1 >> 7) << 11) + (v1 & 127)
        for side, b in ((0, b0), (1, b1)):
            for d in range(DIM):
                j = (g * 2 + side) * DIM + d
                abuf_v[j >> 3, pl.ds((j & 7) * L, L)] = b + d * 128

    copies = [
        pltpu.async_copy(tab_hbm.at[abuf_v.at[j]], dbuf_v.at[j], sem)
        for j in range(128)
    ]

    def sqrt16(s):
        i = plsc.bitcast(s, jnp.int32)
        i = jnp.int32(0x5F3759DF) - (i >> 1)
        y = plsc.bitcast(i, jnp.float32)
        for _ in range(3):
            y = y * (1.5 - 0.5 * s * y * y)
        return jnp.where(s > 0.0, s * y, 0.0)

    acc = jnp.zeros((L,), jnp.float32)
    for g in range(NGRP):
        for r in range(ROWS_PER_GRP):
            copies[g * ROWS_PER_GRP + r].wait()
        s = jnp.zeros((L,), jnp.float32)
        for d in range(DIM):
            a = dbuf_v[g * 4 + (d >> 3), pl.ds((d & 7) * L, L)]
            b = dbuf_v[g * 4 + 2 + (d >> 3), pl.ds((d & 7) * L, L)]
            df = a - b
            s = s + df * df
        dist = sqrt16(s)
        deg = deg_v[pl.ds(g * L, L)]
        acc = acc + deg * dist + jnp.exp(-dist)

    res_v[...] = jnp.full((L,), jnp.sum(acc), jnp.float32)
    pltpu.sync_copy(res_v, out_hbm.at[wid])


def kernel(x, degrees, embs):
    xr = x.astype(jnp.int32).reshape(NW, 8, 128)
    dr = degrees.reshape(NW, BPW)
    mesh = plsc.VectorSubcoreMesh(core_axis_name="c", subcore_axis_name="s")
    tab = pl.kernel(
        _detile_body,
        mesh=mesh,
        out_type=jax.ShapeDtypeStruct(((NFULL + 1) * TILE_W,), jnp.float32),
        scratch_types=[
            pltpu.VMEM((DIM, KCOL * 128), jnp.float32),
            pltpu.VMEM((DIM, KCOL * 128), jnp.float32),
            pltpu.VMEM((WBLK,), jnp.float32),
            pltpu.VMEM((WBLK,), jnp.float32),
            pltpu.VMEM((DIM, 64), jnp.float32),
            pltpu.SemaphoreType.DMA,
            pltpu.SemaphoreType.DMA,
            pltpu.SemaphoreType.DMA,
            pltpu.SemaphoreType.DMA,
        ],
        compiler_params=pltpu.CompilerParams(needs_layout_passes=False),
    )(embs.T, embs.T[:, NFULL * 128:])
    out = pl.kernel(
        _loss_body,
        mesh=mesh,
        out_type=jax.ShapeDtypeStruct((NW, L), jnp.float32),
        scratch_types=[
            pltpu.VMEM((8, 128), jnp.int32),
            pltpu.VMEM((BPW,), jnp.float32),
            pltpu.VMEM((128, 128), jnp.int32),
            pltpu.VMEM((128, 128), jnp.float32),
            pltpu.VMEM((L,), jnp.float32),
            pltpu.SemaphoreType.DMA,
        ],
        compiler_params=pltpu.CompilerParams(needs_layout_passes=False,
                                             use_tc_tiling_on_sc=False),
    )(xr, dr, tab)
    return jnp.sum(out[:, 0])


# speedup vs baseline: 11.8989x; 1.5549x over previous
import jax
import jax.numpy as jnp
from jax import lax
from jax.experimental import pallas as pl
from jax.experimental.pallas import tpu as pltpu
from jax.experimental.pallas import tpu_sc as plsc

DIM = 16
BATCH = 16384
NC = 2
NS = 16
L = 16
NW = NC * NS
BPW = BATCH // NW
NGRP = BPW // L
ROWS_PER_GRP = 2 * DIM * L // 128
DBLOCK = 8000512


NFULL = 1000000 // 128
KCOL = 14
NBLK = 18
TILE_W = 2048
WBLK = KCOL * TILE_W


def _detile_body(embs_t_hbm, tail_t_hbm, out_hbm, rbuf0_v, rbuf1_v,
                 wbufs_sh, tail_v, sem_r0, sem_r1, sem_w0, sem_w1, vsem):
    wid = lax.axis_index("s") * NC + lax.axis_index("c")
    sid = lax.axis_index("s")
    rbufs = (rbuf0_v, rbuf1_v)
    wbufs = (wbufs_sh.at[sid, 0], wbufs_sh.at[sid, 1])
    rsems = (sem_r0, sem_r1)
    wsems = (sem_w0, sem_w1)

    def tile0(blk):
        return jnp.minimum((wid * NBLK + blk) * KCOL, NFULL - KCOL)

    def fire_reads_dyn(blk, par):
        pltpu.async_copy(
            embs_t_hbm.at[:, pl.ds(tile0(blk) * 128, KCOL * 128)],
            rbufs[par], rsems[par])

    def fire_reads(blk):
        fire_reads_dyn(blk, blk % 2)

    def drain(ref, sem):
        pltpu.make_async_copy(out_hbm.at[pl.ds(0, ref.shape[0])]
                              if len(ref.shape) == 1 else
                              embs_t_hbm.at[:, pl.ds(0, ref.shape[1])],
                              ref, sem).wait()

    fire_reads(0)

    def pair_body(pair, carry):
        for b in (0, 1):
            blk = pair * 2 + b
            drain(rbufs[b], rsems[b])

            @pl.when(blk + 1 < NBLK)
            def _():
                fire_reads_dyn(blk + 1, 1 - b)

            @pl.when(blk >= 2)
            def _():
                drain(wbufs[b], wsems[b])

            rbuf = rbufs[b]
            wbuf = wbufs[b]

            def repack(j, c2):
                for d in range(DIM):
                    pltpu.async_copy(
                        rbuf.at[d, pl.ds(j * 128, 128)],
                        wbuf.at[pl.ds(j * TILE_W + d * 128, 128)], vsem)
                return c2

            lax.fori_loop(0, KCOL, repack, jnp.int32(0))
            pltpu.make_async_copy(out_hbm.at[pl.ds(0, WBLK)], wbuf,
                                  vsem).wait()
            pltpu.async_copy(wbuf,
                             out_hbm.at[pl.ds(tile0(blk) * TILE_W, WBLK)],
                             wsems[b])
        return carry

    lax.fori_loop(0, NBLK // 2, pair_body, jnp.int32(0))
    drain(wbufs[0], wsems[0])
    drain(wbufs[1], wsems[1])

    @pl.when(wid == 0)
    def _tail():
        pltpu.sync_copy(tail_t_hbm, tail_v)
        for d in range(DIM):
            pltpu.sync_copy(tail_v.at[d],
                            wbufs_sh.at[sid, 0, pl.ds(d * 128, 64)])
        pltpu.sync_copy(wbufs_sh.at[sid, 0, pl.ds(0, TILE_W)],
                        out_hbm.at[pl.ds(NFULL * TILE_W, TILE_W)])


def _loss_body(x_hbm, deg_hbm, tab_hbm, out_hbm, idx_v, deg_v, abuf_v,
               dbuf_v, res_v, sem):
    wid = lax.axis_index("s") * NC + lax.axis_index("c")
    pltpu.sync_copy(x_hbm.at[wid], idx_v)
    pltpu.sync_copy(deg_hbm.at[wid], deg_v)

    iota = lax.iota(jnp.int32, L)

    for g in range(NGRP):
        k, lp = divmod(g, 4)
        kv = jnp.full((L,), k, jnp.int32)
        p0 = lp * 2 * L + iota * 2
        v0 = plsc.load_gather(idx_v, [kv, p0])
        v1 = plsc.load_gather(idx_v, [kv, p0 + 1])
        b0 = ((v0 >> 7) << 11) + (v0 & 127)
        b1 = ((v1 >> 7) << 11) + (v1 & 127)
        for side, b in ((0, b0), (1, b1)):
            for d in range(DIM):
                j = (g * 2 + side) * DIM + d
                abuf_v[j >> 3, pl.ds((j & 7) * L, L)] = b + d * 128

    copies = [
        pltpu.async_copy(tab_hbm.at[abuf_v.at[j]], dbuf_v.at[j], sem)
        for j in range(128)
    ]

    def sqrt16(s):
        i = plsc.bitcast(s, jnp.int32)
        i = jnp.int32(0x5F3759DF) - (i >> 1)
        y = plsc.bitcast(i, jnp.float32)
        for _ in range(3):
            y = y * (1.5 - 0.5 * s * y * y)
        return jnp.where(s > 0.0, s * y, 0.0)

    acc = jnp.zeros((L,), jnp.float32)
    for g in range(NGRP):
        for r in range(ROWS_PER_GRP):
            copies[g * ROWS_PER_GRP + r].wait()
        s = jnp.zeros((L,), jnp.float32)
        for d in range(DIM):
            a = dbuf_v[g * 4 + (d >> 3), pl.ds((d & 7) * L, L)]
            b = dbuf_v[g * 4 + 2 + (d >> 3), pl.ds((d & 7) * L, L)]
            df = a - b
            s = s + df * df
        dist = sqrt16(s)
        deg = deg_v[pl.ds(g * L, L)]
        acc = acc + deg * dist + jnp.exp(-dist)

    res_v[...] = jnp.full((L,), jnp.sum(acc), jnp.float32)
    pltpu.sync_copy(res_v, out_hbm.at[wid])


def kernel(x, degrees, embs):
    xr = x.astype(jnp.int32).reshape(NW, 8, 128)
    dr = degrees.reshape(NW, BPW)
    mesh = plsc.VectorSubcoreMesh(core_axis_name="c", subcore_axis_name="s")
    tab = pl.kernel(
        _detile_body,
        mesh=mesh,
        out_type=jax.ShapeDtypeStruct(((NFULL + 1) * TILE_W,), jnp.float32),
        scratch_types=[
            pltpu.VMEM((DIM, KCOL * 128), jnp.float32),
            pltpu.VMEM((DIM, KCOL * 128), jnp.float32),
            pltpu.VMEM_SHARED((NS, 2, WBLK), jnp.float32),
            pltpu.VMEM((DIM, 64), jnp.float32),
            pltpu.SemaphoreType.DMA,
            pltpu.SemaphoreType.DMA,
            pltpu.SemaphoreType.DMA,
            pltpu.SemaphoreType.DMA,
            pltpu.SemaphoreType.DMA,
        ],
        compiler_params=pltpu.CompilerParams(needs_layout_passes=False),
    )(embs.T, embs.T[:, NFULL * 128:])
    out = pl.kernel(
        _loss_body,
        mesh=mesh,
        out_type=jax.ShapeDtypeStruct((NW, L), jnp.float32),
        scratch_types=[
            pltpu.VMEM((8, 128), jnp.int32),
            pltpu.VMEM((BPW,), jnp.float32),
            pltpu.VMEM((128, 128), jnp.int32),
            pltpu.VMEM((128, 128), jnp.float32),
            pltpu.VMEM((L,), jnp.float32),
            pltpu.SemaphoreType.DMA,
        ],
        compiler_params=pltpu.CompilerParams(needs_layout_passes=False,
                                             use_tc_tiling_on_sc=False),
    )(xr, dr, tab)
    return jnp.sum(out[:, 0])


# speedup vs baseline: 12.2338x; 1.0281x over previous
import jax
import jax.numpy as jnp
from jax import lax
from jax.experimental import pallas as pl
from jax.experimental.pallas import tpu as pltpu
from jax.experimental.pallas import tpu_sc as plsc

DIM = 16
BATCH = 16384
NC = 2
NS = 16
L = 16
NW = NC * NS
BPW = BATCH // NW
NGRP = BPW // L
ROWS_PER_GRP = 2 * DIM * L // 128
DBLOCK = 8000512


NFULL = 1000000 // 128
KCOL = 14
NBLK = 18
TILE_W = 2048
WBLK = KCOL * TILE_W


def _detile_body(embs_t_hbm, tail_t_hbm, out_hbm, tail_v, tail2_v,
                 wbufs_sh, sem_r0, sem_r1, sem_w0, sem_w1):
    wid = lax.axis_index("s") * NC + lax.axis_index("c")
    sid = lax.axis_index("s")
    wbufs = (wbufs_sh.at[sid, 0], wbufs_sh.at[sid, 1])
    rsems = (sem_r0, sem_r1)
    wsems = (sem_w0, sem_w1)

    def tile0(blk):
        return jnp.minimum((wid * NBLK + blk) * KCOL, NFULL - KCOL)

    def fire_reads_dyn(blk, par):
        t0 = tile0(blk)
        for j in range(KCOL):
            pltpu.async_copy(
                embs_t_hbm.at[:, pl.ds((t0 + j) * 128, 128)],
                wbufs[par].at[j], rsems[par])

    def drain(par, sem):
        pltpu.make_async_copy(out_hbm.at[pl.ds(0, KCOL)], wbufs[par],
                              sem).wait()

    fire_reads_dyn(0, 0)

    def pair_body(pair, carry):
        for b in (0, 1):
            blk = pair * 2 + b
            drain(b, rsems[b])

            @pl.when(blk + 1 < NBLK)
            def _():
                fire_reads_dyn(blk + 1, 1 - b)

            pltpu.async_copy(wbufs[b], out_hbm.at[pl.ds(tile0(blk), KCOL)],
                             wsems[b])

            @pl.when(blk + 2 < NBLK)
            def _():
                drain(b, wsems[b])
        return carry

    lax.fori_loop(0, NBLK // 2, pair_body, jnp.int32(0))
    drain(0, wsems[0])
    drain(1, wsems[1])

    @pl.when(wid == 0)
    def _tail():
        pltpu.sync_copy(tail_t_hbm, tail_v)
        for d in range(DIM):
            for c in range(128 // L):
                tail2_v[d, pl.ds(c * L, L)] = (
                    tail_v[d, pl.ds(64 + c * L, L)] if c < 4 else
                    tail_v[d, pl.ds(c * L, L)])
        pltpu.sync_copy(tail2_v, out_hbm.at[NFULL])


def _loss_body(x_hbm, deg_hbm, tab_hbm, out_hbm, idx_v, deg_v, abuf_v,
               dbuf_v, res_v, sem):
    wid = lax.axis_index("s") * NC + lax.axis_index("c")
    pltpu.sync_copy(x_hbm.at[wid], idx_v)
    pltpu.sync_copy(deg_hbm.at[wid], deg_v)

    iota = lax.iota(jnp.int32, L)

    for g in range(NGRP):
        k, lp = divmod(g, 4)
        kv = jnp.full((L,), k, jnp.int32)
        p0 = lp * 2 * L + iota * 2
        v0 = plsc.load_gather(idx_v, [kv, p0])
        v1 = plsc.load_gather(idx_v, [kv, p0 + 1])
        b0 = ((v0 >> 7) << 11) + (v0 & 127)
        b1 = ((v1 >> 7) << 11) + (v1 & 127)
        for side, b in ((0, b0), (1, b1)):
            for d in range(DIM):
                j = (g * 2 + side) * DIM + d
                abuf_v[j >> 3, pl.ds((j & 7) * L, L)] = b + d * 128

    copies = [
        pltpu.async_copy(tab_hbm.at[abuf_v.at[j]], dbuf_v.at[j], sem)
        for j in range(128)
    ]

    def sqrt16(s):
        i = plsc.bitcast(s, jnp.int32)
        i = jnp.int32(0x5F3759DF) - (i >> 1)
        y = plsc.bitcast(i, jnp.float32)
        for _ in range(3):
            y = y * (1.5 - 0.5 * s * y * y)
        return jnp.where(s > 0.0, s * y, 0.0)

    acc = jnp.zeros((L,), jnp.float32)
    for g in range(NGRP):
        for r in range(ROWS_PER_GRP):
            copies[g * ROWS_PER_GRP + r].wait()
        s = jnp.zeros((L,), jnp.float32)
        for d in range(DIM):
            a = dbuf_v[g * 4 + (d >> 3), pl.ds((d & 7) * L, L)]
            b = dbuf_v[g * 4 + 2 + (d >> 3), pl.ds((d & 7) * L, L)]
            df = a - b
            s = s + df * df
        dist = sqrt16(s)
        deg = deg_v[pl.ds(g * L, L)]
        acc = acc + deg * dist + jnp.exp(-dist)

    res_v[...] = jnp.full((L,), jnp.sum(acc), jnp.float32)
    pltpu.sync_copy(res_v, out_hbm.at[wid])


def kernel(x, degrees, embs):
    xr = x.astype(jnp.int32).reshape(NW, 8, 128)
    dr = degrees.reshape(NW, BPW)
    mesh = plsc.VectorSubcoreMesh(core_axis_name="c", subcore_axis_name="s")
    tab3 = pl.kernel(
        _detile_body,
        mesh=mesh,
        out_type=jax.ShapeDtypeStruct((NFULL + 1, DIM, 128), jnp.float32),
        scratch_types=[
            pltpu.VMEM((DIM, 128), jnp.float32),
            pltpu.VMEM((DIM, 128), jnp.float32),
            pltpu.VMEM_SHARED((NS, 2, KCOL, DIM, 128), jnp.float32),
            pltpu.SemaphoreType.DMA,
            pltpu.SemaphoreType.DMA,
            pltpu.SemaphoreType.DMA,
            pltpu.SemaphoreType.DMA,
        ],
        compiler_params=pltpu.CompilerParams(needs_layout_passes=False),
    )(embs.T, embs.T[:, 1000000 - 128:])
    tab = tab3.reshape((NFULL + 1) * TILE_W)
    out = pl.kernel(
        _loss_body,
        mesh=mesh,
        out_type=jax.ShapeDtypeStruct((NW, L), jnp.float32),
        scratch_types=[
            pltpu.VMEM((8, 128), jnp.int32),
            pltpu.VMEM((BPW,), jnp.float32),
            pltpu.VMEM((128, 128), jnp.int32),
            pltpu.VMEM((128, 128), jnp.float32),
            pltpu.VMEM((L,), jnp.float32),
            pltpu.SemaphoreType.DMA,
        ],
        compiler_params=pltpu.CompilerParams(needs_layout_passes=False,
                                             use_tc_tiling_on_sc=False),
    )(xr, dr, tab)
    return jnp.sum(out[:, 0])
